# baseline (speedup 1.0000x reference)
import functools

import jax
import jax.numpy as jnp
from jax import lax
from jax.experimental import pallas as pl
from jax.experimental.pallas import tpu as pltpu
from jax.experimental.pallas import tpu_sc as plsc

B = 4096
L = 200
D = 64
V = 100004
VC = 100000
NC = 2
NS = 16
NW = NC * NS
RPW = B // NW
LP = 208
SPLIT = 128

SLAB = 512
NSLABS = VC // SLAB
SLAB_STRIDE = 520

_SC_PARAMS = pltpu.CompilerParams(
    use_tc_tiling_on_sc=False, needs_layout_passes=False)
_MESH = plsc.VectorSubcoreMesh(core_axis_name="c", subcore_axis_name="s")


@functools.partial(
    pl.kernel,
    out_type=jax.ShapeDtypeStruct((VC, D), jnp.bfloat16),
    mesh=_MESH,
    compiler_params=_SC_PARAMS,
    scratch_types=[
        pltpu.VMEM((D, SLAB_STRIDE), jnp.float32),
        pltpu.VMEM((SLAB, D), jnp.bfloat16),
    ],
)
def _transpose_to_bf16(tt_h, out_h, inb, outb):
    wid = lax.axis_index("s") * NC + lax.axis_index("c")

    ridx = [lax.iota(jnp.int32, 16) + 16 * c for c in range(4)]

    def do_slab(e_base, n):
        pltpu.sync_copy(tt_h.at[:, pl.ds(e_base, n)], inb.at[:, pl.ds(0, n)])

        def body(e, carry):
            eidx = jnp.broadcast_to(e.astype(jnp.int32), (16,))
            g = [plsc.load_gather(inb, [ridx[c], eidx]) for c in range(4)]
            outb[e, pl.ds(0, 32)] = plsc.pack(
                g[0], g[1], format=plsc.PackFormat.INTERLEAVED)
            outb[e, pl.ds(32, 32)] = plsc.pack(
                g[2], g[3], format=plsc.PackFormat.INTERLEAVED)
            return carry

        lax.fori_loop(0, n, body, 0, unroll=4)
        pltpu.sync_copy(outb.at[pl.ds(0, n)], out_h.at[pl.ds(e_base, n)])

    def slabs(j, carry):
        s = wid + NW * j

        @pl.when(s < NSLABS)
        def _():
            do_slab(s * SLAB, SLAB)

        return carry

    lax.fori_loop(0, (NSLABS + NW - 1) // NW, slabs, 0)

    @pl.when(wid == NW - 1)
    def _():
        do_slab(NSLABS * SLAB, VC - NSLABS * SLAB)


@functools.partial(
    pl.kernel,
    out_type=jax.ShapeDtypeStruct((B, D), jnp.float32),
    mesh=_MESH,
    compiler_params=_SC_PARAMS,
    scratch_types=[
        pltpu.VMEM((RPW, LP), jnp.int32),
        pltpu.VMEM((L, D), jnp.bfloat16),
        pltpu.VMEM((L, D), jnp.bfloat16),
        pltpu.VMEM((RPW, D), jnp.float32),
        pltpu.SMEM((RPW,), jnp.float32),
        pltpu.SemaphoreType.DMA,
        pltpu.SemaphoreType.DMA,
    ],
)
def _avg_embed(code_h, table_h, out_h, idx_v, buf0, buf1, out_v, cnts,
               sem0, sem1):
    wid = lax.axis_index("s") * NC + lax.axis_index("c")
    base = wid * RPW

    pltpu.sync_copy(code_h.at[pl.ds(base, RPW)], idx_v.at[:, pl.ds(0, L)])

    lane = lax.iota(jnp.int32, 16)

    def prep_row(r, carry):
        def pbody(k, cv):
            v = idx_v[r, pl.ds(k * 16, 16)]
            cv = cv + jnp.where(v != 0, 1.0, 0.0).astype(jnp.float32)
            idx_v[r, pl.ds(k * 16, 16)] = jnp.where(v >= VC, 0, v)
            return cv

        cv = lax.fori_loop(0, 12, pbody, jnp.zeros((16,), jnp.float32),
                           unroll=4)
        vtail = idx_v[r, pl.ds(192, 16)]
        cv = cv + jnp.where((vtail != 0) & (lane < 8), 1.0, 0.0).astype(
            jnp.float32)
        idx_v[r, pl.ds(192, 16)] = jnp.where(vtail >= VC, 0, vtail)
        cnts[r] = jnp.sum(cv)
        return carry

    lax.fori_loop(0, RPW, prep_row, 0)

    bufs = (buf0, buf1)
    sems = (sem0, sem1)

    def start(r, b):
        pltpu.async_copy(
            table_h.at[idx_v.at[r, pl.ds(0, SPLIT)]],
            bufs[b].at[pl.ds(0, SPLIT)],
            sems[b],
        )
        pltpu.async_copy(
            table_h.at[idx_v.at[r, pl.ds(SPLIT, L - SPLIT)]],
            bufs[b].at[pl.ds(SPLIT, L - SPLIT)],
            sems[b],
        )

    def wait(b):
        pltpu.make_async_copy(table_h.at[pl.ds(0, L)], bufs[b], sems[b]).wait()

    def reduce_row(buf, r):
        cnt = jnp.broadcast_to(cnts[r], (16,))

        def sbody(l, accs):
            a0, a1, a2, a3 = accs
            e0, o0 = plsc.unpack(buf[l, pl.ds(0, 32)],
                                 format=plsc.PackFormat.INTERLEAVED)
            e1, o1 = plsc.unpack(buf[l, pl.ds(32, 32)],
                                 format=plsc.PackFormat.INTERLEAVED)
            return (a0 + e0, a1 + o0, a2 + e1, a3 + o1)

        z = jnp.zeros((16,), jnp.float32)
        a0, a1, a2, a3 = lax.fori_loop(0, L, sbody, (z, z, z, z), unroll=8)
        out_v[r, pl.ds(0, 16)] = a0 / cnt
        out_v[r, pl.ds(16, 16)] = a1 / cnt
        out_v[r, pl.ds(32, 16)] = a2 / cnt
        out_v[r, pl.ds(48, 16)] = a3 / cnt

    start(0, 0)

    def gbody(g, carry):
        r0 = 2 * g
        start(r0 + 1, 1)
        wait(0)
        reduce_row(buf0, r0)

        @pl.when(g < RPW // 2 - 1)
        def _():
            start(r0 + 2, 0)

        wait(1)
        reduce_row(buf1, r0 + 1)
        return carry

    lax.fori_loop(0, RPW // 2, gbody, 0)

    pltpu.sync_copy(out_v, out_h.at[pl.ds(base, RPW)])


def kernel(code, code_table):
    code = code.astype(jnp.int32)
    table_bf16 = _transpose_to_bf16(code_table.T)
    out = _avg_embed(code, table_bf16)

    denom = jnp.sum((code != 0).astype(jnp.float32), axis=1)
    fix = jnp.zeros((B, D), jnp.float32)
    for k in range(VC, V):
        occ = jnp.sum((code == k).astype(jnp.float32), axis=1)
        fix = fix + occ[:, None] * code_table[k][None, :]
    return out + fix / denom[:, None]


# speedup vs baseline: 2.4600x; 2.4600x over previous
import functools

import jax
import jax.numpy as jnp
from jax import lax
from jax.experimental import pallas as pl
from jax.experimental.pallas import tpu as pltpu
from jax.experimental.pallas import tpu_sc as plsc

B = 4096
L = 200
D = 64
NC = 2
NS = 16
NW = NC * NS
RPW = B // NW
LP = 208
SPLIT = 128


@functools.partial(
    pl.kernel,
    out_type=jax.ShapeDtypeStruct((B, D), jnp.float32),
    mesh=plsc.VectorSubcoreMesh(core_axis_name="c", subcore_axis_name="s"),
    compiler_params=pltpu.CompilerParams(
        use_tc_tiling_on_sc=False, needs_layout_passes=False),
    scratch_types=[
        pltpu.VMEM((RPW, LP), jnp.int32),
        pltpu.VMEM((L, D), jnp.float32),
        pltpu.VMEM((L, D), jnp.float32),
        pltpu.VMEM((RPW, D), jnp.float32),
        pltpu.SemaphoreType.DMA,
        pltpu.SemaphoreType.DMA,
    ],
)
def _avg_embed(code_h, table_h, out_h, idx_v, buf0, buf1, out_v, sem0, sem1):
    wid = lax.axis_index("s") * NC + lax.axis_index("c")
    base = wid * RPW

    pltpu.sync_copy(code_h.at[pl.ds(base, RPW)], idx_v.at[:, pl.ds(0, L)])

    bufs = (buf0, buf1)
    sems = (sem0, sem1)

    def start(r, b):
        pltpu.async_copy(
            table_h.at[idx_v.at[r, pl.ds(0, SPLIT)]],
            bufs[b].at[pl.ds(0, SPLIT)],
            sems[b],
        )
        pltpu.async_copy(
            table_h.at[idx_v.at[r, pl.ds(SPLIT, L - SPLIT)]],
            bufs[b].at[pl.ds(SPLIT, L - SPLIT)],
            sems[b],
        )

    def wait(b):
        pltpu.make_async_copy(table_h.at[pl.ds(0, L)], bufs[b], sems[b]).wait()

    lane = lax.iota(jnp.int32, 16)

    def reduce_row(buf, r):
        def cbody(k, cv):
            v = idx_v[r, pl.ds(k * 16, 16)]
            return cv + jnp.where(v != 0, 1.0, 0.0).astype(jnp.float32)

        cv = lax.fori_loop(0, 12, cbody, jnp.zeros((16,), jnp.float32),
                           unroll=4)
        vtail = idx_v[r, pl.ds(192, 16)]
        cv = cv + jnp.where((vtail != 0) & (lane < 8), 1.0, 0.0).astype(
            jnp.float32)
        cnt = jnp.broadcast_to(jnp.sum(cv), (16,))

        def sbody(l, accs):
            a0, a1, a2, a3 = accs
            return (
                a0 + buf[l, pl.ds(0, 16)],
                a1 + buf[l, pl.ds(16, 16)],
                a2 + buf[l, pl.ds(32, 16)],
                a3 + buf[l, pl.ds(48, 16)],
            )

        z = jnp.zeros((16,), jnp.float32)
        a0, a1, a2, a3 = lax.fori_loop(0, L, sbody, (z, z, z, z), unroll=8)
        out_v[r, pl.ds(0, 16)] = a0 / cnt
        out_v[r, pl.ds(16, 16)] = a1 / cnt
        out_v[r, pl.ds(32, 16)] = a2 / cnt
        out_v[r, pl.ds(48, 16)] = a3 / cnt

    start(0, 0)

    def gbody(g, carry):
        r0 = 2 * g
        start(r0 + 1, 1)
        wait(0)
        reduce_row(buf0, r0)

        @pl.when(g < RPW // 2 - 1)
        def _():
            start(r0 + 2, 0)

        wait(1)
        reduce_row(buf1, r0 + 1)
        return carry

    lax.fori_loop(0, RPW // 2, gbody, 0)

    pltpu.sync_copy(out_v, out_h.at[pl.ds(base, RPW)])


def kernel(code, code_table):
    return _avg_embed(code.astype(jnp.int32), code_table)
